# baseline (speedup 1.0000x reference)
import functools

import jax
import jax.numpy as jnp
from jax import lax
from jax.experimental import pallas as pl
from jax.experimental.pallas import tpu as pltpu
from jax.experimental.pallas import tpu_sc as plsc

N_NODES = 10000
NP = 10240
N_EDGES = 160000
FEAT = 256
HALF = 128
NS = 16
RPT = NP // NS
RSTAGE = 128

_mesh = plsc.VectorSubcoreMesh(core_axis_name="c", subcore_axis_name="s")


CHUNK2 = 128
NCH2 = 80
EPAD = NS * NCH2 * CHUNK2
LAST_REAL = (N_EDGES - (NS - 1) * NCH2 * CHUNK2) // CHUNK2


@functools.partial(
    pl.kernel,
    out_type=[jax.ShapeDtypeStruct((NP,), jnp.float32),
              jax.ShapeDtypeStruct((NP,), jnp.float32)],
    mesh=_mesh,
    scratch_types=[
        pltpu.VMEM((NCH2, CHUNK2), jnp.int32),
        pltpu.VMEM((CHUNK2,), jnp.float32),
        pltpu.VMEM((RPT,), jnp.float32),
        pltpu.VMEM_SHARED((NP,), jnp.float32),
        pltpu.SemaphoreType.DMA,
    ],
)
def _degrees(src3_hbm, dst3_hbm, osrc, odst, idx_v, ones_v, stage_v, acc_sh,
             sem):
    cid = lax.axis_index("c")
    sid = lax.axis_index("s")

    def _zrow(i, c):
        stage_v[pl.ds(i * 16, 16)] = jnp.zeros((16,), jnp.float32)
        return c
    lax.fori_loop(jnp.int32(0), jnp.int32(RPT // 16), _zrow, jnp.int32(0))
    for j in range(CHUNK2 // 16):
        ones_v[pl.ds(j * 16, 16)] = jnp.ones((16,), jnp.float32)

    rbase = pl.multiple_of(sid * RPT, 8)
    pltpu.sync_copy(stage_v, acc_sh.at[pl.ds(rbase, RPT)])

    @pl.when(cid == 0)
    def _():
        pltpu.sync_copy(src3_hbm.at[sid], idx_v)

    @pl.when(cid == 1)
    def _():
        pltpu.sync_copy(dst3_hbm.at[sid], idx_v)

    plsc.subcore_barrier()

    nch = jnp.where(sid == NS - 1, jnp.int32(LAST_REAL), jnp.int32(NCH2))

    def _body(j, c):
        pltpu.async_copy(ones_v, acc_sh.at[idx_v.at[j]], sem, add=True)
        return c
    lax.fori_loop(jnp.int32(0), nch, _body, jnp.int32(0))

    def _drain(j, c):
        pltpu.make_async_copy(
            ones_v, acc_sh.at[idx_v.at[jnp.int32(0)]], sem).wait()
        return c
    lax.fori_loop(jnp.int32(0), nch, _drain, jnp.int32(0))

    plsc.subcore_barrier()

    pltpu.sync_copy(acc_sh.at[pl.ds(rbase, RPT)], stage_v)

    @pl.when(cid == 0)
    def _():
        pltpu.sync_copy(stage_v, osrc.at[pl.ds(rbase, RPT)])

    @pl.when(cid == 1)
    def _():
        pltpu.sync_copy(stage_v, odst.at[pl.ds(rbase, RPT)])


NHA = NCH2 // 2


@functools.partial(
    pl.kernel,
    out_type=[jax.ShapeDtypeStruct((NP, HALF), jnp.float32),
              jax.ShapeDtypeStruct((NP, HALF), jnp.float32)],
    mesh=_mesh,
    scratch_types=[
        pltpu.VMEM((NHA, CHUNK2), jnp.int32),
        pltpu.VMEM((NHA, CHUNK2), jnp.int32),
        pltpu.VMEM((CHUNK2, HALF), jnp.float32),
        pltpu.VMEM((CHUNK2, HALF), jnp.float32),
        pltpu.VMEM_SHARED((NP, HALF), jnp.float32),
        pltpu.SemaphoreType.DMA,
        pltpu.SemaphoreType.DMA,
    ],
)
def _edge_agg(xl_hbm, xr_hbm, src3_hbm, dst3_hbm, outl, outr,
              src_v, dst_v, rows0_v, rows1_v, acc_sh, sem0, sem1):
    cid = lax.axis_index("c")
    sid = lax.axis_index("s")

    def _zrow(i, c):
        for j in range(HALF // 16):
            rows0_v[i, pl.ds(j * 16, 16)] = jnp.zeros((16,), jnp.float32)
        return c
    lax.fori_loop(jnp.int32(0), jnp.int32(CHUNK2), _zrow, jnp.int32(0))
    for t in range(RPT // CHUNK2):
        start = pl.multiple_of(sid * RPT + t * CHUNK2, 8)
        pltpu.sync_copy(rows0_v, acc_sh.at[pl.ds(start, CHUNK2)])

    plsc.subcore_barrier()

    def _run(x_hbm):
        for h in range(2):
            pltpu.sync_copy(src3_hbm.at[sid, pl.ds(h * NHA, NHA)], src_v)
            pltpu.sync_copy(dst3_hbm.at[sid, pl.ds(h * NHA, NHA)], dst_v)
            pltpu.async_copy(x_hbm.at[src_v.at[jnp.int32(0)]], rows0_v, sem0)

            def _body(i, c):
                j0 = 2 * i
                j1 = j0 + 1
                pltpu.async_copy(x_hbm.at[src_v.at[j1]], rows1_v, sem1)
                pltpu.make_async_copy(
                    x_hbm.at[src_v.at[j0]], rows0_v, sem0).wait()
                pltpu.sync_copy(rows0_v, acc_sh.at[dst_v.at[j0]], add=True)

                @pl.when(j0 + 2 < NHA)
                def _():
                    pltpu.async_copy(x_hbm.at[src_v.at[j0 + 2]], rows0_v, sem0)

                pltpu.make_async_copy(
                    x_hbm.at[src_v.at[j1]], rows1_v, sem1).wait()
                pltpu.sync_copy(rows1_v, acc_sh.at[dst_v.at[j1]], add=True)
                return c
            lax.fori_loop(jnp.int32(0), jnp.int32(NHA // 2), _body,
                          jnp.int32(0))

    @pl.when(cid == 0)
    def _():
        _run(xl_hbm)

    @pl.when(cid == 1)
    def _():
        _run(xr_hbm)

    plsc.subcore_barrier()

    for t in range(RPT // CHUNK2):
        start = pl.multiple_of(sid * RPT + t * CHUNK2, 8)
        sl = pl.ds(start, CHUNK2)
        pltpu.sync_copy(acc_sh.at[sl], rows0_v)

        @pl.when(cid == 0)
        def _():
            pltpu.sync_copy(rows0_v, outl.at[sl])

        @pl.when(cid == 1)
        def _():
            pltpu.sync_copy(rows0_v, outr.at[sl])


@functools.partial(
    pl.kernel,
    out_type=jax.ShapeDtypeStruct((N_NODES, FEAT), jnp.float32),
    mesh=_mesh,
    scratch_types=[
        pltpu.VMEM((NHA, CHUNK2), jnp.int32),
        pltpu.VMEM((NHA, CHUNK2), jnp.int32),
        pltpu.VMEM((CHUNK2, HALF), jnp.float32),
        pltpu.VMEM((CHUNK2, HALF), jnp.float32),
        pltpu.VMEM((RPT,), jnp.float32),
        pltpu.VMEM((HALF,), jnp.float32),
        pltpu.VMEM_SHARED((NP, HALF), jnp.float32),
        pltpu.SemaphoreType.DMA,
        pltpu.SemaphoreType.DMA,
    ],
)
def _agg_fin(xl_hbm, xr_hbm, src3_hbm, dst3_hbm, nd_hbm, b2_hbm, out,
             src_v, dst_v, rows0_v, rows1_v, nd_v, b_v, acc_sh, sem0, sem1):
    cid = lax.axis_index("c")
    sid = lax.axis_index("s")

    def _zrow(i, c):
        for j in range(HALF // 16):
            rows0_v[i, pl.ds(j * 16, 16)] = jnp.zeros((16,), jnp.float32)
        return c
    lax.fori_loop(jnp.int32(0), jnp.int32(CHUNK2), _zrow, jnp.int32(0))
    for t in range(RPT // CHUNK2):
        start = pl.multiple_of(sid * RPT + t * CHUNK2, 8)
        pltpu.sync_copy(rows0_v, acc_sh.at[pl.ds(start, CHUNK2)])

    rbase = pl.multiple_of(sid * RPT, 8)
    pltpu.sync_copy(b2_hbm.at[cid], b_v)

    @pl.when(sid < NS - 1)
    def _():
        pltpu.sync_copy(nd_hbm.at[pl.ds(rbase, RPT)], nd_v)

    @pl.when(sid == NS - 1)
    def _():
        pltpu.sync_copy(nd_hbm.at[pl.ds(rbase, N_NODES - (NS - 1) * RPT)],
                        nd_v.at[pl.ds(0, N_NODES - (NS - 1) * RPT)])

    plsc.subcore_barrier()

    def _run(x_hbm):
        for h in range(2):
            pltpu.sync_copy(src3_hbm.at[sid, pl.ds(h * NHA, NHA)], src_v)
            pltpu.sync_copy(dst3_hbm.at[sid, pl.ds(h * NHA, NHA)], dst_v)
            pltpu.async_copy(x_hbm.at[src_v.at[jnp.int32(0)]], rows0_v, sem0)

            def _body(i, c):
                j0 = 2 * i
                j1 = j0 + 1
                pltpu.async_copy(x_hbm.at[src_v.at[j1]], rows1_v, sem1)
                pltpu.make_async_copy(
                    x_hbm.at[src_v.at[j0]], rows0_v, sem0).wait()
                pltpu.sync_copy(rows0_v, acc_sh.at[dst_v.at[j0]], add=True)

                @pl.when(j0 + 2 < NHA)
                def _():
                    pltpu.async_copy(x_hbm.at[src_v.at[j0 + 2]], rows0_v, sem0)

                pltpu.make_async_copy(
                    x_hbm.at[src_v.at[j1]], rows1_v, sem1).wait()
                pltpu.sync_copy(rows1_v, acc_sh.at[dst_v.at[j1]], add=True)
                return c
            lax.fori_loop(jnp.int32(0), jnp.int32(NHA // 2), _body,
                          jnp.int32(0))

    @pl.when(cid == 0)
    def _():
        _run(xl_hbm)

    @pl.when(cid == 1)
    def _():
        _run(xr_hbm)

    plsc.subcore_barrier()

    col = pl.multiple_of(cid * HALF, 128)
    for t in range(RPT // CHUNK2):
        start = pl.multiple_of(rbase + t * CHUNK2, 8)

        def _do(nrows):
            pltpu.sync_copy(acc_sh.at[pl.ds(start, nrows)],
                            rows0_v.at[pl.ds(0, nrows)])

            def _fblk(k, c):
                nds = nd_v[pl.ds(t * CHUNK2 + k * 16, 16)]
                for r in range(16):
                    i = k * 16 + r
                    sc = nds[r]
                    for j in range(HALF // 16):
                        cs = pl.ds(j * 16, 16)
                        rows0_v[i, cs] = jnp.maximum(
                            rows0_v[i, cs] * sc + b_v[cs], 0.0)
                return c
            lax.fori_loop(jnp.int32(0), jnp.int32(nrows // 16), _fblk,
                          jnp.int32(0))
            pltpu.sync_copy(rows0_v.at[pl.ds(0, nrows)],
                            out.at[pl.ds(start, nrows), pl.ds(col, HALF)])

        @pl.when(start + CHUNK2 <= N_NODES)
        def _():
            _do(CHUNK2)

        @pl.when(jnp.logical_and(start < N_NODES, start + CHUNK2 > N_NODES))
        def _():
            _do(N_NODES % CHUNK2)


_BM = 2000


def _norm(d):
    return jnp.where(d > 0.0, lax.rsqrt(d), 0.0)


def _mm1_body(h_ref, w_ref, ds_ref, ol_ref, or_ref):
    x = jnp.dot(h_ref[...], w_ref[...],
                preferred_element_type=jnp.float32,
                precision=lax.Precision.DEFAULT)
    x = x * _norm(ds_ref[...])
    ol_ref[...] = x[:, :HALF]
    or_ref[...] = x[:, HALF:]


def _mid_body(al_ref, ar_ref, ds_ref, dd_ref, b_ref, w_ref, ol_ref, or_ref,
              nd_ref):
    nd = _norm(dd_ref[...])
    nd_ref[...] = nd
    agg = jnp.concatenate([al_ref[...], ar_ref[...]], axis=1)
    t = jnp.maximum(agg * nd + b_ref[...], 0.0)
    t = t * _norm(ds_ref[...])
    x = jnp.dot(t, w_ref[...],
                preferred_element_type=jnp.float32,
                precision=lax.Precision.DEFAULT)
    ol_ref[...] = x[:, :HALF]
    or_ref[...] = x[:, HALF:]


_row_spec = pl.BlockSpec((_BM, FEAT), lambda i: (i, jnp.int32(0)))
_half_spec = pl.BlockSpec((_BM, HALF), lambda i: (i, jnp.int32(0)))
_deg_spec = pl.BlockSpec((_BM, 1), lambda i: (i, jnp.int32(0)))
_w_spec = pl.BlockSpec((FEAT, FEAT), lambda i: (jnp.int32(0), jnp.int32(0)))
_b_spec = pl.BlockSpec((1, FEAT), lambda i: (jnp.int32(0), jnp.int32(0)))
_grid = (N_NODES // _BM,)

_mm1 = pl.pallas_call(
    _mm1_body,
    grid=_grid,
    in_specs=[_row_spec, _w_spec, _deg_spec],
    out_specs=[_half_spec, _half_spec],
    out_shape=[jax.ShapeDtypeStruct((N_NODES, HALF), jnp.float32),
               jax.ShapeDtypeStruct((N_NODES, HALF), jnp.float32)],
)

_mid = pl.pallas_call(
    _mid_body,
    grid=_grid,
    in_specs=[_half_spec, _half_spec, _deg_spec, _deg_spec, _b_spec, _w_spec],
    out_specs=[_half_spec, _half_spec, _deg_spec],
    out_shape=[jax.ShapeDtypeStruct((N_NODES, HALF), jnp.float32),
               jax.ShapeDtypeStruct((N_NODES, HALF), jnp.float32),
               jax.ShapeDtypeStruct((N_NODES, 1), jnp.float32)],
)

def kernel(h, edge_index, W1, b1, W2, b2):
    src = edge_index[0].astype(jnp.int32)
    dst = edge_index[1].astype(jnp.int32)
    h = h.astype(jnp.float32)

    fill = jnp.arange(EPAD - N_EDGES, dtype=jnp.int32)
    src_p = jnp.concatenate([src, fill % jnp.int32(N_NODES)])
    dst_p = jnp.concatenate(
        [dst, jnp.int32(N_NODES) + fill % jnp.int32(NP - N_NODES)])
    src3 = src_p.reshape(NS, NCH2, CHUNK2)
    dst3 = dst_p.reshape(NS, NCH2, CHUNK2)

    deg_src, deg_dst = _degrees(src3, dst3)
    ds2 = deg_src.reshape(NP, 1)
    dd2 = deg_dst.reshape(NP, 1)
    b1r = b1.astype(jnp.float32).reshape(1, FEAT)

    x1l, x1r = _mm1(h, W1.astype(jnp.float32), ds2)
    a1l, a1r = _edge_agg(x1l, x1r, src3, dst3)
    x2l, x2r, ndc = _mid(a1l, a1r, ds2, dd2, b1r, W2.astype(jnp.float32))
    return _agg_fin(x2l, x2r, src3, dst3, ndc.reshape(N_NODES),
                    b2.astype(jnp.float32).reshape(2, HALF))


# speedup vs baseline: 1.1024x; 1.1024x over previous
import functools

import jax
import jax.numpy as jnp
from jax import lax
from jax.experimental import pallas as pl
from jax.experimental.pallas import tpu as pltpu
from jax.experimental.pallas import tpu_sc as plsc

N_NODES = 10000
NP = 10240
N_EDGES = 160000
FEAT = 256
HALF = 128
NS = 16
RPT = NP // NS
RSTAGE = 128

_mesh = plsc.VectorSubcoreMesh(core_axis_name="c", subcore_axis_name="s")


CHUNK2 = 128
NCH2 = 80
EPAD = NS * NCH2 * CHUNK2
LAST_REAL = (N_EDGES - (NS - 1) * NCH2 * CHUNK2) // CHUNK2


@functools.partial(
    pl.kernel,
    out_type=[jax.ShapeDtypeStruct((NP,), jnp.float32),
              jax.ShapeDtypeStruct((NP,), jnp.float32)],
    mesh=_mesh,
    scratch_types=[
        pltpu.VMEM((NCH2, CHUNK2), jnp.int32),
        pltpu.VMEM((CHUNK2,), jnp.float32),
        pltpu.VMEM((RPT,), jnp.float32),
        pltpu.VMEM_SHARED((NP,), jnp.float32),
        pltpu.SemaphoreType.DMA,
    ],
)
def _degrees(src3_hbm, dst3_hbm, osrc, odst, idx_v, ones_v, stage_v, acc_sh,
             sem):
    cid = lax.axis_index("c")
    sid = lax.axis_index("s")

    def _zrow(i, c):
        stage_v[pl.ds(i * 16, 16)] = jnp.zeros((16,), jnp.float32)
        return c
    lax.fori_loop(jnp.int32(0), jnp.int32(RPT // 16), _zrow, jnp.int32(0))
    for j in range(CHUNK2 // 16):
        ones_v[pl.ds(j * 16, 16)] = jnp.ones((16,), jnp.float32)

    rbase = pl.multiple_of(sid * RPT, 8)
    pltpu.sync_copy(stage_v, acc_sh.at[pl.ds(rbase, RPT)])

    @pl.when(cid == 0)
    def _():
        pltpu.sync_copy(src3_hbm.at[sid], idx_v)

    @pl.when(cid == 1)
    def _():
        pltpu.sync_copy(dst3_hbm.at[sid], idx_v)

    plsc.subcore_barrier()

    nch = jnp.where(sid == NS - 1, jnp.int32(LAST_REAL), jnp.int32(NCH2))

    def _body(j, c):
        pltpu.async_copy(ones_v, acc_sh.at[idx_v.at[j]], sem, add=True)
        return c
    lax.fori_loop(jnp.int32(0), nch, _body, jnp.int32(0))

    def _drain(j, c):
        pltpu.make_async_copy(
            ones_v, acc_sh.at[idx_v.at[jnp.int32(0)]], sem).wait()
        return c
    lax.fori_loop(jnp.int32(0), nch, _drain, jnp.int32(0))

    plsc.subcore_barrier()

    pltpu.sync_copy(acc_sh.at[pl.ds(rbase, RPT)], stage_v)

    @pl.when(cid == 0)
    def _():
        pltpu.sync_copy(stage_v, osrc.at[pl.ds(rbase, RPT)])

    @pl.when(cid == 1)
    def _():
        pltpu.sync_copy(stage_v, odst.at[pl.ds(rbase, RPT)])


NHA = NCH2 // 2


@functools.partial(
    pl.kernel,
    out_type=[jax.ShapeDtypeStruct((NP, HALF), jnp.float32),
              jax.ShapeDtypeStruct((NP, HALF), jnp.float32)],
    mesh=_mesh,
    scratch_types=[
        pltpu.VMEM((NHA, CHUNK2), jnp.int32),
        pltpu.VMEM((NHA, CHUNK2), jnp.int32),
        pltpu.VMEM((CHUNK2, HALF), jnp.float32),
        pltpu.VMEM((CHUNK2, HALF), jnp.float32),
        pltpu.VMEM_SHARED((NP, HALF), jnp.float32),
        pltpu.SemaphoreType.DMA,
        pltpu.SemaphoreType.DMA,
    ],
)
def _edge_agg(xl_hbm, xr_hbm, src3_hbm, dst3_hbm, outl, outr,
              src_v, dst_v, rows0_v, rows1_v, acc_sh, sem0, sem1):
    cid = lax.axis_index("c")
    sid = lax.axis_index("s")

    def _zrow(i, c):
        for j in range(HALF // 16):
            rows0_v[i, pl.ds(j * 16, 16)] = jnp.zeros((16,), jnp.float32)
        return c
    lax.fori_loop(jnp.int32(0), jnp.int32(CHUNK2), _zrow, jnp.int32(0))
    for t in range(RPT // CHUNK2):
        start = pl.multiple_of(sid * RPT + t * CHUNK2, 8)
        pltpu.sync_copy(rows0_v, acc_sh.at[pl.ds(start, CHUNK2)])

    plsc.subcore_barrier()

    def _run(x_hbm):
        for h in range(2):
            pltpu.sync_copy(src3_hbm.at[sid, pl.ds(h * NHA, NHA)], src_v)
            pltpu.sync_copy(dst3_hbm.at[sid, pl.ds(h * NHA, NHA)], dst_v)
            pltpu.async_copy(x_hbm.at[src_v.at[jnp.int32(0)]], rows0_v, sem0)

            def _body(i, c):
                j0 = 2 * i
                j1 = j0 + 1
                pltpu.async_copy(x_hbm.at[src_v.at[j1]], rows1_v, sem1)
                pltpu.make_async_copy(
                    x_hbm.at[src_v.at[j0]], rows0_v, sem0).wait()
                pltpu.sync_copy(rows0_v, acc_sh.at[dst_v.at[j0]], add=True)

                @pl.when(j0 + 2 < NHA)
                def _():
                    pltpu.async_copy(x_hbm.at[src_v.at[j0 + 2]], rows0_v, sem0)

                pltpu.make_async_copy(
                    x_hbm.at[src_v.at[j1]], rows1_v, sem1).wait()
                pltpu.sync_copy(rows1_v, acc_sh.at[dst_v.at[j1]], add=True)
                return c
            lax.fori_loop(jnp.int32(0), jnp.int32(NHA // 2), _body,
                          jnp.int32(0))

    @pl.when(cid == 0)
    def _():
        _run(xl_hbm)

    @pl.when(cid == 1)
    def _():
        _run(xr_hbm)

    plsc.subcore_barrier()

    for t in range(RPT // CHUNK2):
        start = pl.multiple_of(sid * RPT + t * CHUNK2, 8)
        sl = pl.ds(start, CHUNK2)
        pltpu.sync_copy(acc_sh.at[sl], rows0_v)

        @pl.when(cid == 0)
        def _():
            pltpu.sync_copy(rows0_v, outl.at[sl])

        @pl.when(cid == 1)
        def _():
            pltpu.sync_copy(rows0_v, outr.at[sl])


_BM = 2000


def _norm(d):
    return jnp.where(d > 0.0, lax.rsqrt(d), 0.0)


def _mm1_body(h_ref, w_ref, ds_ref, ol_ref, or_ref):
    x = jnp.dot(h_ref[...], w_ref[...],
                preferred_element_type=jnp.float32,
                precision=lax.Precision.DEFAULT)
    x = x * _norm(ds_ref[...])
    ol_ref[...] = x[:, :HALF]
    or_ref[...] = x[:, HALF:]


def _mid_body(al_ref, ar_ref, ds_ref, dd_ref, b_ref, w_ref, ol_ref, or_ref):
    agg = jnp.concatenate([al_ref[...], ar_ref[...]], axis=1)
    t = jnp.maximum(agg * _norm(dd_ref[...]) + b_ref[...], 0.0)
    t = t * _norm(ds_ref[...])
    x = jnp.dot(t, w_ref[...],
                preferred_element_type=jnp.float32,
                precision=lax.Precision.DEFAULT)
    ol_ref[...] = x[:, :HALF]
    or_ref[...] = x[:, HALF:]


def _fin_body(al_ref, ar_ref, dd_ref, b_ref, o_ref):
    agg = jnp.concatenate([al_ref[...], ar_ref[...]], axis=1)
    o_ref[...] = jnp.maximum(agg * _norm(dd_ref[...]) + b_ref[...], 0.0)


_row_spec = pl.BlockSpec((_BM, FEAT), lambda i: (i, jnp.int32(0)))
_half_spec = pl.BlockSpec((_BM, HALF), lambda i: (i, jnp.int32(0)))
_deg_spec = pl.BlockSpec((_BM, 1), lambda i: (i, jnp.int32(0)))
_w_spec = pl.BlockSpec((FEAT, FEAT), lambda i: (jnp.int32(0), jnp.int32(0)))
_b_spec = pl.BlockSpec((1, FEAT), lambda i: (jnp.int32(0), jnp.int32(0)))
_grid = (N_NODES // _BM,)

_mm1 = pl.pallas_call(
    _mm1_body,
    grid=_grid,
    in_specs=[_row_spec, _w_spec, _deg_spec],
    out_specs=[_half_spec, _half_spec],
    out_shape=[jax.ShapeDtypeStruct((N_NODES, HALF), jnp.float32),
               jax.ShapeDtypeStruct((N_NODES, HALF), jnp.float32)],
)

_mid = pl.pallas_call(
    _mid_body,
    grid=_grid,
    in_specs=[_half_spec, _half_spec, _deg_spec, _deg_spec, _b_spec, _w_spec],
    out_specs=[_half_spec, _half_spec],
    out_shape=[jax.ShapeDtypeStruct((N_NODES, HALF), jnp.float32),
               jax.ShapeDtypeStruct((N_NODES, HALF), jnp.float32)],
)

_fin = pl.pallas_call(
    _fin_body,
    grid=_grid,
    in_specs=[_half_spec, _half_spec, _deg_spec, _b_spec],
    out_specs=_row_spec,
    out_shape=jax.ShapeDtypeStruct((N_NODES, FEAT), jnp.float32),
)


def kernel(h, edge_index, W1, b1, W2, b2):
    src = edge_index[0].astype(jnp.int32)
    dst = edge_index[1].astype(jnp.int32)
    h = h.astype(jnp.float32)

    fill = jnp.arange(EPAD - N_EDGES, dtype=jnp.int32)
    src_p = jnp.concatenate([src, fill % jnp.int32(N_NODES)])
    dst_p = jnp.concatenate(
        [dst, jnp.int32(N_NODES) + fill % jnp.int32(NP - N_NODES)])
    src3 = src_p.reshape(NS, NCH2, CHUNK2)
    dst3 = dst_p.reshape(NS, NCH2, CHUNK2)

    deg_src, deg_dst = _degrees(src3, dst3)
    ds2 = deg_src.reshape(NP, 1)
    dd2 = deg_dst.reshape(NP, 1)
    b1r = b1.astype(jnp.float32).reshape(1, FEAT)
    b2r = b2.astype(jnp.float32).reshape(1, FEAT)

    x1l, x1r = _mm1(h, W1.astype(jnp.float32), ds2)
    a1l, a1r = _edge_agg(x1l, x1r, src3, dst3)
    x2l, x2r = _mid(a1l, a1r, ds2, dd2, b1r, W2.astype(jnp.float32))
    a2l, a2r = _edge_agg(x2l, x2r, src3, dst3)
    return _fin(a2l, a2r, dd2, b2r)


# speedup vs baseline: 1.1264x; 1.0217x over previous
import functools

import jax
import jax.numpy as jnp
from jax import lax
from jax.experimental import pallas as pl
from jax.experimental.pallas import tpu as pltpu
from jax.experimental.pallas import tpu_sc as plsc

N_NODES = 10000
NP = 10240
N_EDGES = 160000
FEAT = 256
HALF = 128
NS = 16
RPT = NP // NS

_mesh = plsc.VectorSubcoreMesh(core_axis_name="c", subcore_axis_name="s")


CHUNK2 = 128
NCH2 = 80
EPAD = NS * NCH2 * CHUNK2
LAST_REAL = (N_EDGES - (NS - 1) * NCH2 * CHUNK2) // CHUNK2


@functools.partial(
    pl.kernel,
    out_type=[jax.ShapeDtypeStruct((NP,), jnp.float32),
              jax.ShapeDtypeStruct((NP,), jnp.float32)],
    mesh=_mesh,
    scratch_types=[
        pltpu.VMEM((NCH2, CHUNK2), jnp.int32),
        pltpu.VMEM((CHUNK2,), jnp.float32),
        pltpu.VMEM((RPT,), jnp.float32),
        pltpu.VMEM_SHARED((NP,), jnp.float32),
        pltpu.SemaphoreType.DMA,
    ],
)
def _degrees(src3_hbm, dst3_hbm, osrc, odst, idx_v, ones_v, stage_v, acc_sh,
             sem):
    cid = lax.axis_index("c")
    sid = lax.axis_index("s")

    def _zrow(i, c):
        stage_v[pl.ds(i * 16, 16)] = jnp.zeros((16,), jnp.float32)
        return c
    lax.fori_loop(jnp.int32(0), jnp.int32(RPT // 16), _zrow, jnp.int32(0))
    for j in range(CHUNK2 // 16):
        ones_v[pl.ds(j * 16, 16)] = jnp.ones((16,), jnp.float32)

    rbase = pl.multiple_of(sid * RPT, 8)
    pltpu.sync_copy(stage_v, acc_sh.at[pl.ds(rbase, RPT)])

    @pl.when(cid == 0)
    def _():
        pltpu.sync_copy(src3_hbm.at[sid], idx_v)

    @pl.when(cid == 1)
    def _():
        pltpu.sync_copy(dst3_hbm.at[sid], idx_v)

    plsc.subcore_barrier()

    nch = jnp.where(sid == NS - 1, jnp.int32(LAST_REAL), jnp.int32(NCH2))

    def _body(j, c):
        pltpu.async_copy(ones_v, acc_sh.at[idx_v.at[j]], sem, add=True)
        return c
    lax.fori_loop(jnp.int32(0), nch, _body, jnp.int32(0))

    def _drain(j, c):
        pltpu.make_async_copy(
            ones_v, acc_sh.at[idx_v.at[jnp.int32(0)]], sem).wait()
        return c
    lax.fori_loop(jnp.int32(0), nch, _drain, jnp.int32(0))

    plsc.subcore_barrier()

    pltpu.sync_copy(acc_sh.at[pl.ds(rbase, RPT)], stage_v)

    @pl.when(cid == 0)
    def _():
        pltpu.sync_copy(stage_v, osrc.at[pl.ds(rbase, RPT)])

    @pl.when(cid == 1)
    def _():
        pltpu.sync_copy(stage_v, odst.at[pl.ds(rbase, RPT)])


NHA = NCH2 // 2


@functools.partial(
    pl.kernel,
    out_type=[jax.ShapeDtypeStruct((NP, HALF), jnp.float32),
              jax.ShapeDtypeStruct((NP, HALF), jnp.float32)],
    mesh=_mesh,
    scratch_types=[
        pltpu.VMEM((NHA, CHUNK2), jnp.int32),
        pltpu.VMEM((NHA, CHUNK2), jnp.int32),
        pltpu.VMEM((CHUNK2, HALF), jnp.float32),
        pltpu.VMEM((CHUNK2, HALF), jnp.float32),
        pltpu.VMEM_SHARED((NP, HALF), jnp.float32),
        pltpu.SemaphoreType.DMA,
        pltpu.SemaphoreType.DMA,
    ],
)
def _edge_agg(xl_hbm, xr_hbm, src3_hbm, dst3_hbm, outl, outr,
              src_v, dst_v, rows0_v, rows1_v, acc_sh, sem0, sem1):
    cid = lax.axis_index("c")
    sid = lax.axis_index("s")

    def _zrow(i, c):
        for j in range(HALF // 16):
            rows0_v[i, pl.ds(j * 16, 16)] = jnp.zeros((16,), jnp.float32)
        return c
    pltpu.async_copy(src3_hbm.at[sid, pl.ds(0, NHA)], src_v, sem0)
    pltpu.async_copy(dst3_hbm.at[sid, pl.ds(0, NHA)], dst_v, sem1)
    lax.fori_loop(jnp.int32(0), jnp.int32(CHUNK2), _zrow, jnp.int32(0))
    for t in range(RPT // CHUNK2):
        start = pl.multiple_of(sid * RPT + t * CHUNK2, 8)
        pltpu.sync_copy(rows0_v, acc_sh.at[pl.ds(start, CHUNK2)])
    pltpu.make_async_copy(src3_hbm.at[sid, pl.ds(0, NHA)], src_v, sem0).wait()
    pltpu.make_async_copy(dst3_hbm.at[sid, pl.ds(0, NHA)], dst_v, sem1).wait()

    plsc.subcore_barrier()

    def _run(x_hbm):
        for h in range(2):
            if h:
                pltpu.sync_copy(src3_hbm.at[sid, pl.ds(h * NHA, NHA)], src_v)
                pltpu.sync_copy(dst3_hbm.at[sid, pl.ds(h * NHA, NHA)], dst_v)
            pltpu.async_copy(x_hbm.at[src_v.at[jnp.int32(0)]], rows0_v, sem0)

            def _body(i, c):
                j0 = 2 * i
                j1 = j0 + 1
                pltpu.async_copy(x_hbm.at[src_v.at[j1]], rows1_v, sem1)
                pltpu.make_async_copy(
                    x_hbm.at[src_v.at[j0]], rows0_v, sem0).wait()
                pltpu.sync_copy(rows0_v, acc_sh.at[dst_v.at[j0]], add=True)

                @pl.when(j0 + 2 < NHA)
                def _():
                    pltpu.async_copy(x_hbm.at[src_v.at[j0 + 2]], rows0_v, sem0)

                pltpu.make_async_copy(
                    x_hbm.at[src_v.at[j1]], rows1_v, sem1).wait()
                pltpu.sync_copy(rows1_v, acc_sh.at[dst_v.at[j1]], add=True)
                return c
            lax.fori_loop(jnp.int32(0), jnp.int32(NHA // 2), _body,
                          jnp.int32(0))

    @pl.when(cid == 0)
    def _():
        _run(xl_hbm)

    @pl.when(cid == 1)
    def _():
        _run(xr_hbm)

    plsc.subcore_barrier()

    def _dsl(t):
        return pl.ds(pl.multiple_of(sid * RPT + t * CHUNK2, 8), CHUNK2)

    bufs = (rows0_v, rows1_v)
    sems = (sem0, sem1)
    pltpu.async_copy(acc_sh.at[_dsl(0)], rows0_v, sem0)
    for t in range(RPT // CHUNK2):
        buf, sm = bufs[t % 2], sems[t % 2]
        pltpu.make_async_copy(acc_sh.at[_dsl(t)], buf, sm).wait()
        if t + 1 < RPT // CHUNK2:
            pltpu.async_copy(acc_sh.at[_dsl(t + 1)], bufs[(t + 1) % 2],
                             sems[(t + 1) % 2])

        @pl.when(cid == 0)
        def _():
            pltpu.sync_copy(buf, outl.at[_dsl(t)])

        @pl.when(cid == 1)
        def _():
            pltpu.sync_copy(buf, outr.at[_dsl(t)])


_BM = 2000


def _norm(d):
    return jnp.where(d > 0.0, lax.rsqrt(d), 0.0)


def _mm1_body(h_ref, w_ref, ds_ref, ol_ref, or_ref):
    x = jnp.dot(h_ref[...], w_ref[...],
                preferred_element_type=jnp.float32,
                precision=lax.Precision.DEFAULT)
    x = x * _norm(ds_ref[...])
    ol_ref[...] = x[:, :HALF]
    or_ref[...] = x[:, HALF:]


def _mid_body(al_ref, ar_ref, ds_ref, dd_ref, b_ref, w_ref, ol_ref, or_ref):
    agg = jnp.concatenate([al_ref[...], ar_ref[...]], axis=1)
    t = jnp.maximum(agg * _norm(dd_ref[...]) + b_ref[...], 0.0)
    t = t * _norm(ds_ref[...])
    x = jnp.dot(t, w_ref[...],
                preferred_element_type=jnp.float32,
                precision=lax.Precision.DEFAULT)
    ol_ref[...] = x[:, :HALF]
    or_ref[...] = x[:, HALF:]


def _fin_body(al_ref, ar_ref, dd_ref, b_ref, o_ref):
    agg = jnp.concatenate([al_ref[...], ar_ref[...]], axis=1)
    o_ref[...] = jnp.maximum(agg * _norm(dd_ref[...]) + b_ref[...], 0.0)


_row_spec = pl.BlockSpec((_BM, FEAT), lambda i: (i, jnp.int32(0)))
_half_spec = pl.BlockSpec((_BM, HALF), lambda i: (i, jnp.int32(0)))
_deg_spec = pl.BlockSpec((_BM, 1), lambda i: (i, jnp.int32(0)))
_w_spec = pl.BlockSpec((FEAT, FEAT), lambda i: (jnp.int32(0), jnp.int32(0)))
_b_spec = pl.BlockSpec((1, FEAT), lambda i: (jnp.int32(0), jnp.int32(0)))
_grid = (N_NODES // _BM,)

_mm1 = pl.pallas_call(
    _mm1_body,
    grid=_grid,
    in_specs=[_row_spec, _w_spec, _deg_spec],
    out_specs=[_half_spec, _half_spec],
    out_shape=[jax.ShapeDtypeStruct((N_NODES, HALF), jnp.float32),
               jax.ShapeDtypeStruct((N_NODES, HALF), jnp.float32)],
)

_mid = pl.pallas_call(
    _mid_body,
    grid=_grid,
    in_specs=[_half_spec, _half_spec, _deg_spec, _deg_spec, _b_spec, _w_spec],
    out_specs=[_half_spec, _half_spec],
    out_shape=[jax.ShapeDtypeStruct((N_NODES, HALF), jnp.float32),
               jax.ShapeDtypeStruct((N_NODES, HALF), jnp.float32)],
)

_fin = pl.pallas_call(
    _fin_body,
    grid=_grid,
    in_specs=[_half_spec, _half_spec, _deg_spec, _b_spec],
    out_specs=_row_spec,
    out_shape=jax.ShapeDtypeStruct((N_NODES, FEAT), jnp.float32),
)


def kernel(h, edge_index, W1, b1, W2, b2):
    src = edge_index[0].astype(jnp.int32)
    dst = edge_index[1].astype(jnp.int32)
    h = h.astype(jnp.float32)

    fill = jnp.arange(EPAD - N_EDGES, dtype=jnp.int32)
    src_p = jnp.concatenate([src, fill % jnp.int32(N_NODES)])
    dst_p = jnp.concatenate(
        [dst, jnp.int32(N_NODES) + fill % jnp.int32(NP - N_NODES)])
    src3 = src_p.reshape(NS, NCH2, CHUNK2)
    dst3 = dst_p.reshape(NS, NCH2, CHUNK2)

    deg_src, deg_dst = _degrees(src3, dst3)
    ds2 = deg_src.reshape(NP, 1)
    dd2 = deg_dst.reshape(NP, 1)
    b1r = b1.astype(jnp.float32).reshape(1, FEAT)
    b2r = b2.astype(jnp.float32).reshape(1, FEAT)

    x1l, x1r = _mm1(h, W1.astype(jnp.float32), ds2)
    a1l, a1r = _edge_agg(x1l, x1r, src3, dst3)
    x2l, x2r = _mid(a1l, a1r, ds2, dd2, b1r, W2.astype(jnp.float32))
    a2l, a2r = _edge_agg(x2l, x2r, src3, dst3)
    return _fin(a2l, a2r, dd2, b2r)
